# baseline (speedup 1.0000x reference)
import jax
import jax.numpy as jnp
import numpy as np
from jax.experimental import pallas as pl
from jax.experimental.pallas import tpu as pltpu

_NB_SYMB = 14
_FFT = 4096
_SPACING = 4
_NPIL = _FFT // _SPACING
_BBLK = 32
_KCHUNK = 512
_NK = _FFT // _KCHUNK
_XW = 256
_QPC = _KCHUNK // _SPACING


def _freq_interp_blocks() -> np.ndarray:
    e = np.zeros((_NPIL, _FFT), np.float32)
    for k in range(_FFT):
        q = k // _SPACING
        if q >= _NPIL - 1:
            e[_NPIL - 1, k] = 1.0
        else:
            w = (k % _SPACING) / _SPACING
            e[q, k] = 1.0 - w
            e[q + 1, k] = w
    blocks = np.zeros((_NK, _XW, _KCHUNK), np.float32)
    for j in range(_NK):
        s = min(j * _QPC, _NPIL - _XW)
        blocks[j] = e[s:s + _XW, j * _KCHUNK:(j + 1) * _KCHUNK]
    return blocks


_E2 = _freq_interp_blocks()
_TNORM = np.clip((np.arange(_NB_SYMB) - 2.0) / 9.0, 0.0, 1.0).astype(np.float32)


def _body(x_ref, e_ref, o_ref):
    for j in range(_NK):
        sj = min(j * _QPC, _NPIL - _XW)
        ej = e_ref[j]
        x0 = x_ref[:, sj:sj + _XW]
        x1 = x_ref[:, _NPIL + sj:_NPIL + sj + _XW]
        hf0 = jax.lax.dot(
            x0, ej,
            precision=jax.lax.Precision.DEFAULT,
            preferred_element_type=jnp.float32,
        )
        hf1 = jax.lax.dot(
            x1, ej,
            precision=jax.lax.Precision.DEFAULT,
            preferred_element_type=jnp.float32,
        )
        d = hf1 - hf0
        ksl = slice(j * _KCHUNK, (j + 1) * _KCHUNK)
        for s in range(_NB_SYMB):
            t = float(_TNORM[s])
            if t == 0.0:
                o_ref[s, :, ksl] = hf0
            elif t == 1.0:
                o_ref[s, :, ksl] = hf1
            else:
                o_ref[s, :, ksl] = hf0 + t * d


@jax.jit
def kernel(inputs):
    b = inputs.shape[0]
    e2 = jnp.asarray(_E2)
    out3 = pl.pallas_call(
        _body,
        grid=(b // _BBLK,),
        in_specs=[
            pl.BlockSpec((_BBLK, 2 * _NPIL), lambda i: (i, 0)),
            pl.BlockSpec((_NK, _XW, _KCHUNK), lambda i: (0, 0, 0)),
        ],
        out_specs=pl.BlockSpec((_NB_SYMB, _BBLK, _FFT), lambda i: (0, i, 0)),
        out_shape=jax.ShapeDtypeStruct((_NB_SYMB, b, _FFT), inputs.dtype),
        compiler_params=pltpu.CompilerParams(
            dimension_semantics=("arbitrary",),
        ),
    )(inputs, e2)
    return jnp.transpose(out3, (1, 0, 2))


# speedup vs baseline: 1.0457x; 1.0457x over previous
import jax
import jax.numpy as jnp
import numpy as np
from jax.experimental import pallas as pl
from jax.experimental.pallas import tpu as pltpu

_NB_SYMB = 14
_FFT = 4096
_SPACING = 4
_NPIL = _FFT // _SPACING
_BBLK = 32
_KCHUNK = 512
_NK = _FFT // _KCHUNK
_XW = 136
_QPC = _KCHUNK // _SPACING


def _freq_interp_blocks() -> np.ndarray:
    e = np.zeros((_NPIL, _FFT), np.float32)
    for k in range(_FFT):
        q = k // _SPACING
        if q >= _NPIL - 1:
            e[_NPIL - 1, k] = 1.0
        else:
            w = (k % _SPACING) / _SPACING
            e[q, k] = 1.0 - w
            e[q + 1, k] = w
    blocks = np.zeros((_NK, _XW, _KCHUNK), np.float32)
    for j in range(_NK):
        s = min(j * _QPC, _NPIL - _XW)
        blocks[j] = e[s:s + _XW, j * _KCHUNK:(j + 1) * _KCHUNK]
    return blocks


_E2 = _freq_interp_blocks()
_TNORM = np.clip((np.arange(_NB_SYMB) - 2.0) / 9.0, 0.0, 1.0).astype(np.float32)


def _body(x_ref, e_ref, o_ref):
    for j in range(_NK):
        sj = min(j * _QPC, _NPIL - _XW)
        ej = e_ref[j]
        x0 = x_ref[:, sj:sj + _XW]
        x1 = x_ref[:, _NPIL + sj:_NPIL + sj + _XW]
        hf0 = jax.lax.dot(
            x0, ej,
            precision=jax.lax.Precision.DEFAULT,
            preferred_element_type=jnp.float32,
        )
        hf1 = jax.lax.dot(
            x1, ej,
            precision=jax.lax.Precision.DEFAULT,
            preferred_element_type=jnp.float32,
        )
        d = hf1 - hf0
        ksl = slice(j * _KCHUNK, (j + 1) * _KCHUNK)
        for s in range(_NB_SYMB):
            t = float(_TNORM[s])
            if t == 0.0:
                o_ref[s, :, ksl] = hf0
            elif t == 1.0:
                o_ref[s, :, ksl] = hf1
            else:
                o_ref[s, :, ksl] = hf0 + t * d


@jax.jit
def kernel(inputs):
    b = inputs.shape[0]
    e2 = jnp.asarray(_E2)
    out3 = pl.pallas_call(
        _body,
        grid=(b // _BBLK,),
        in_specs=[
            pl.BlockSpec((_BBLK, 2 * _NPIL), lambda i: (i, 0)),
            pl.BlockSpec((_NK, _XW, _KCHUNK), lambda i: (0, 0, 0)),
        ],
        out_specs=pl.BlockSpec((_NB_SYMB, _BBLK, _FFT), lambda i: (0, i, 0)),
        out_shape=jax.ShapeDtypeStruct((_NB_SYMB, b, _FFT), inputs.dtype),
        compiler_params=pltpu.CompilerParams(
            dimension_semantics=("arbitrary",),
        ),
    )(inputs, e2)
    return jnp.transpose(out3, (1, 0, 2))
